# baseline (speedup 1.0000x reference)
import functools

import jax
import jax.numpy as jnp
from jax import lax
from jax.experimental import pallas as pl
from jax.experimental.pallas import tpu as pltpu
from jax.experimental.pallas import tpu_sc as plsc

_DIM = 32
_N_EMBED = 8192
_N_TOKENS = 64 * 1024
_NC = 2
_NS = 16
_NW = _NC * _NS
_B_PER_W = _N_TOKENS // _NW
_CHUNK = 512
_G = 128

_mesh = plsc.VectorSubcoreMesh(core_axis_name="c", subcore_axis_name="s")


@functools.partial(
    pl.kernel,
    out_type=(
        jax.ShapeDtypeStruct((_N_TOKENS * _DIM,), jnp.float32),
        jax.ShapeDtypeStruct((_NW * 16,), jnp.float32),
    ),
    mesh=_mesh,
    scratch_types=[
        pltpu.VMEM((_B_PER_W,), jnp.int32),
        pltpu.VMEM((_CHUNK, _DIM), jnp.float32),
        pltpu.VMEM((_CHUNK * _DIM,), jnp.float32),
        pltpu.VMEM((_CHUNK * _DIM,), jnp.float32),
        pltpu.VMEM((16,), jnp.float32),
        pltpu.SemaphoreType.DMA,
    ],
    compiler_params=pltpu.CompilerParams(use_tc_tiling_on_sc=False),
)
def _vq_lookup(inp_hbm, lab_hbm, emb_hbm, quant_hbm, part_hbm,
               idx_v, rows_v, inp_v, outf_v, acc_v, sem):
    wid = lax.axis_index("s") * _NC + lax.axis_index("c")
    base = wid * _B_PER_W
    acc_v[...] = jnp.zeros((16,), jnp.float32)
    pltpu.sync_copy(lab_hbm.at[pl.ds(base, _B_PER_W)], idx_v)
    for c in range(_B_PER_W // _CHUNK):
        off = c * _CHUNK
        copies = [
            pltpu.async_copy(
                emb_hbm.at[idx_v.at[pl.ds(off + j * _G, _G)]],
                rows_v.at[pl.ds(j * _G, _G)],
                sem,
            )
            for j in range(_CHUNK // _G)
        ]
        pltpu.sync_copy(inp_hbm.at[pl.ds((base + off) * _DIM, _CHUNK * _DIM)],
                        inp_v)
        for cp in copies:
            cp.wait()

        @pl.loop(0, _CHUNK)
        def _(i):
            r0 = rows_v[i, pl.ds(0, 16)]
            r1 = rows_v[i, pl.ds(16, 16)]
            x0 = inp_v[pl.ds(i * _DIM, 16)]
            x1 = inp_v[pl.ds(i * _DIM + 16, 16)]
            outf_v[pl.ds(i * _DIM, 16)] = r0
            outf_v[pl.ds(i * _DIM + 16, 16)] = r1
            d0 = r0 - x0
            d1 = r1 - x1
            acc_v[...] = acc_v[...] + d0 * d0 + d1 * d1

        pltpu.sync_copy(outf_v,
                        quant_hbm.at[pl.ds((base + off) * _DIM, _CHUNK * _DIM)])

    pltpu.sync_copy(acc_v, part_hbm.at[pl.ds(wid * 16, 16)])


def kernel(input, labels, embed):
    inp_flat = input.reshape(_N_TOKENS * _DIM)
    emb_t = embed.T
    quant, partials = _vq_lookup(inp_flat, labels, emb_t)
    quantize = quant.reshape(input.shape)
    diff = jnp.sum(partials) / jnp.float32(_N_TOKENS * _DIM)
    embed_ind = labels.reshape(input.shape[:-1])
    return quantize, diff, embed_ind


# speedup vs baseline: 1.1303x; 1.1303x over previous
import functools

import jax
import jax.numpy as jnp
from jax import lax
from jax.experimental import pallas as pl
from jax.experimental.pallas import tpu as pltpu
from jax.experimental.pallas import tpu_sc as plsc

_DIM = 32
_N_EMBED = 8192
_B = 64
_T = 1024
_N_TOKENS = _B * _T
_NC = 2
_NS = 16
_NW = _NC * _NS
_B_PER_W = _N_TOKENS // _NW
_CHUNK = 512
_G = 128
_NCHUNK = _B_PER_W // _CHUNK

_mesh = plsc.VectorSubcoreMesh(core_axis_name="c", subcore_axis_name="s")


@functools.partial(
    pl.kernel,
    out_type=jax.ShapeDtypeStruct((_N_TOKENS, _DIM), jnp.float32),
    mesh=_mesh,
    scratch_types=[
        pltpu.VMEM((_B_PER_W,), jnp.int32),
        pltpu.VMEM((_CHUNK, _DIM), jnp.float32),
        pltpu.VMEM((_CHUNK, _DIM), jnp.float32),
        pltpu.SemaphoreType.DMA,
        pltpu.SemaphoreType.DMA,
        pltpu.SemaphoreType.DMA,
        pltpu.SemaphoreType.DMA,
    ],
    compiler_params=pltpu.CompilerParams(use_tc_tiling_on_sc=False),
)
def _vq_gather(lab_hbm, emb_hbm, quant_hbm,
               idx_v, rows0, rows1, g0, g1, s0, s1):
    wid = lax.axis_index("s") * _NC + lax.axis_index("c")
    base = wid * _B_PER_W
    pltpu.sync_copy(lab_hbm.at[pl.ds(base, _B_PER_W)], idx_v)

    bufs = [rows0, rows1]
    gsems = [g0, g1]
    ssems = [s0, s1]

    def fire(c):
        buf, sem = bufs[c % 2], gsems[c % 2]
        return [
            pltpu.async_copy(
                emb_hbm.at[idx_v.at[pl.ds(c * _CHUNK + j * _G, _G)]],
                buf.at[pl.ds(j * _G, _G)],
                sem,
            )
            for j in range(_CHUNK // _G)
        ]

    gathers = {0: fire(0)}
    stores = {}
    for c in range(_NCHUNK):
        if c + 1 < _NCHUNK:
            if c - 1 >= 0:
                stores[c - 1].wait()
            gathers[c + 1] = fire(c + 1)
        for cp in gathers[c]:
            cp.wait()
        stores[c] = pltpu.async_copy(
            bufs[c % 2],
            quant_hbm.at[pl.ds(base + c * _CHUNK, _CHUNK)],
            ssems[c % 2],
        )
    stores[_NCHUNK - 2].wait()
    stores[_NCHUNK - 1].wait()


@functools.partial(
    pl.pallas_call,
    grid=(_B,),
    in_specs=[
        pl.BlockSpec((_T, _DIM), lambda i: (i, 0)),
        pl.BlockSpec((1, _DIM, _T), lambda i: (i, 0, 0)),
    ],
    out_specs=[
        pl.BlockSpec((1, _DIM, _T), lambda i: (i, 0, 0)),
        pl.BlockSpec((1, 1, 128), lambda i: (i, 0, 0)),
    ],
    out_shape=[
        jax.ShapeDtypeStruct((_B, _DIM, _T), jnp.float32),
        jax.ShapeDtypeStruct((_B, 1, 128), jnp.float32),
    ],
)
def _transpose_mse(q_ref, x_ref, out_ref, p_ref):
    q_t = q_ref[...].T
    out_ref[0] = q_t
    d = q_t - x_ref[0]
    p_ref[...] = jnp.broadcast_to(jnp.sum(d * d), (1, 1, 128))


def kernel(input, labels, embed):
    emb_t = embed.T
    quant_tm = _vq_gather(labels, emb_t)
    x_t = input.transpose(0, 2, 1)
    out_t, partials = _transpose_mse(quant_tm, x_t)
    quantize = out_t.transpose(0, 2, 1)
    diff = jnp.sum(partials[:, 0, 0]) / jnp.float32(_N_TOKENS * _DIM)
    embed_ind = labels.reshape(_B, _T)
    return quantize, diff, embed_ind


# speedup vs baseline: 1.3110x; 1.1598x over previous
import functools

import jax
import jax.numpy as jnp
from jax import lax
from jax.experimental import pallas as pl
from jax.experimental.pallas import tpu as pltpu
from jax.experimental.pallas import tpu_sc as plsc

_DIM = 32
_N_EMBED = 8192
_B = 64
_T = 1024
_N_TOKENS = _B * _T
_NC = 2
_NS = 16
_NW = _NC * _NS
_DG = 4
_DPG = _DIM // _DG
_BG = _NW // _DG
_BPG = _B // _BG

_mesh = plsc.VectorSubcoreMesh(core_axis_name="c", subcore_axis_name="s")


@functools.partial(
    pl.kernel,
    out_type=jax.ShapeDtypeStruct((_B, _DIM, _T), jnp.float32),
    mesh=_mesh,
    scratch_types=[
        pltpu.VMEM((_DPG, _N_EMBED), jnp.float32),
        pltpu.VMEM((_T,), jnp.int32),
        pltpu.VMEM((_T,), jnp.int32),
        pltpu.VMEM((_DPG, _T), jnp.float32),
        pltpu.VMEM((_DPG, _T), jnp.float32),
        pltpu.SemaphoreType.DMA,
        pltpu.SemaphoreType.DMA,
        pltpu.SemaphoreType.DMA,
        pltpu.SemaphoreType.DMA,
        pltpu.SemaphoreType.DMA,
    ],
    compiler_params=pltpu.CompilerParams(use_tc_tiling_on_sc=False,
                                         needs_layout_passes=False),
)
def _vq_lookup(lab_hbm, emb_hbm, q_hbm,
               tab_v, idx0, idx1, out0, out1, tsem, i0, i1, o0, o1):
    wid = lax.axis_index("s") * _NC + lax.axis_index("c")
    bg = wid // _DG
    dg = wid % _DG
    b0 = bg * _BPG
    d0 = dg * _DPG

    tab_cp = pltpu.async_copy(emb_hbm.at[pl.ds(d0, _DPG), :], tab_v, tsem)
    idxs = [idx0, idx1]
    isems = [i0, i1]
    outs = [out0, out1]
    osems = [o0, o1]

    def load_idx(k):
        return pltpu.async_copy(
            lab_hbm.at[pl.ds((b0 + k) * _T, _T)], idxs[k % 2], isems[k % 2])

    icopies = {0: load_idx(0)}
    ocopies = {}
    tab_cp.wait()
    for k in range(_BPG):
        if k + 1 < _BPG:
            icopies[k + 1] = load_idx(k + 1)
        icopies[k].wait()
        if k - 2 >= 0:
            ocopies[k - 2].wait()
        idx_v = idxs[k % 2]
        out_v = outs[k % 2]

        @pl.loop(0, _T, step=16)
        def _(t0):
            iv = idx_v[pl.ds(t0, 16)]
            for dl in range(_DPG):
                val = plsc.load_gather(
                    tab_v, [jnp.full((16,), dl, jnp.int32), iv])
                out_v[dl, pl.ds(t0, 16)] = val

        ocopies[k] = pltpu.async_copy(
            out_v, q_hbm.at[b0 + k, pl.ds(d0, _DPG), :], osems[k % 2])
    ocopies[_BPG - 2].wait()
    ocopies[_BPG - 1].wait()


@functools.partial(
    pl.pallas_call,
    grid=(_B,),
    in_specs=[
        pl.BlockSpec((_T * _DIM // 128, 128), lambda i: (i, 0)),
        pl.BlockSpec((1, _DIM, _T), lambda i: (i, 0, 0)),
    ],
    out_specs=[
        pl.BlockSpec((1, _DIM, _T), lambda i: (i, 0, 0)),
        pl.BlockSpec((1, 1, 128), lambda i: (i, 0, 0)),
    ],
    out_shape=[
        jax.ShapeDtypeStruct((_B, _DIM, _T), jnp.float32),
        jax.ShapeDtypeStruct((_B, 1, 128), jnp.float32),
    ],
)
def _retile_mse(q_ref, x_ref, out_ref, p_ref):
    q = q_ref[...].reshape(_DIM, _T)
    out_ref[0] = q
    d = q - x_ref[0]
    p_ref[...] = jnp.broadcast_to(jnp.sum(d * d), (1, 1, 128))


def kernel(input, labels, embed):
    q3 = _vq_lookup(labels, embed)
    q_flat = q3.reshape(_N_TOKENS * _DIM // 128, 128)
    x_t = input.transpose(0, 2, 1)
    out_t, partials = _retile_mse(q_flat, x_t)
    quantize = out_t.transpose(0, 2, 1)
    diff = jnp.sum(partials[:, 0, 0]) / jnp.float32(_N_TOKENS * _DIM)
    embed_ind = labels.reshape(_B, _T)
    return quantize, diff, embed_ind
